# baseline (speedup 1.0000x reference)
import functools

import jax
import jax.numpy as jnp
from jax import lax
from jax.experimental import pallas as pl
from jax.experimental.pallas import tpu as pltpu
from jax.experimental.pallas import tpu_sc as plsc

N_SRC = 10000
N_DST = 10000
N_PAD = 10240
E = 320000
D = 128
HD = D // 2
NC = 2
NS = 16
EPT = E // NS
BLK = 80
PBLK = BLK // 2
NBLK = EPT // BLK
RPT = N_PAD // NS
CW = 16
VL = 16


def _mm_bias_body(x_ref, w_ref, b_ref, o_ref):
    o_ref[...] = (
        jnp.dot(x_ref[...], w_ref[...], preferred_element_type=jnp.float32)
        + b_ref[...]
    )


def _project_bias(x, w, b, bm):
    m, k = x.shape
    n = w.shape[1]
    return pl.pallas_call(
        _mm_bias_body,
        grid=(m // bm,),
        in_specs=[
            pl.BlockSpec((bm, k), lambda i: (i, 0)),
            pl.BlockSpec((k, n), lambda i: (0, 0)),
            pl.BlockSpec((1, n), lambda i: (0, 0)),
        ],
        out_specs=pl.BlockSpec((bm, n), lambda i: (i, 0)),
        out_shape=jax.ShapeDtypeStruct((m, n), jnp.float32),
    )(x, w, b.reshape(1, -1))


def _proj_split_body(x_ref, w_ref, o_ref):
    res = jnp.dot(x_ref[...], w_ref[...], preferred_element_type=jnp.float32)
    o_ref[0] = res[:, :HD]
    o_ref[1] = res[:, HD:]


def _project_split(x, w, bm):
    m, k = x.shape
    return pl.pallas_call(
        _proj_split_body,
        grid=(m // bm,),
        in_specs=[
            pl.BlockSpec((bm, k), lambda i: (i, 0)),
            pl.BlockSpec((k, D), lambda i: (0, 0)),
        ],
        out_specs=pl.BlockSpec((NC, bm, HD), lambda i: (0, i, 0)),
        out_shape=jax.ShapeDtypeStruct((NC, m, HD), jnp.float32),
    )(x, w)


def _sc_aggregate(src_proj, edge_projp, src_idx, dst_idx):
    mesh = plsc.VectorSubcoreMesh(core_axis_name="c", subcore_axis_name="s")

    @functools.partial(
        pl.kernel,
        mesh=mesh,
        compiler_params=pltpu.CompilerParams(use_tc_tiling_on_sc=False),
        out_type=[
            jax.ShapeDtypeStruct((NC, N_PAD, HD), jnp.float32),
            jax.ShapeDtypeStruct((N_PAD, CW), jnp.float32),
        ],
        scratch_types=[
            pltpu.VMEM((NBLK, BLK), jnp.int32),
            pltpu.VMEM((NBLK, BLK), jnp.int32),
            pltpu.VMEM((2, BLK, D), jnp.float32),
            pltpu.VMEM((2, PBLK, D), jnp.float32),
            pltpu.VMEM((BLK, HD), jnp.float32),
            pltpu.VMEM((BLK, CW), jnp.float32),
            pltpu.VMEM((BLK, CW), jnp.float32),
            pltpu.SemaphoreType.DMA,
            pltpu.SemaphoreType.DMA,
            pltpu.SemaphoreType.DMA,
            pltpu.SemaphoreType.DMA,
            pltpu.VMEM_SHARED((N_PAD, HD), jnp.float32),
            pltpu.VMEM_SHARED((N_PAD, CW), jnp.float32),
        ],
    )
    def body(src_proj_hbm, edge_proj_hbm, sidx_hbm, didx_hbm,
             agg_out, cnt_out,
             sidx_v, didx_v, rows_v, ep_v, h_v, cz_v, ones_v,
             sem_g0, sem_g1, sem_e0, sem_e1, agg_sh, cnt_sh):
        cid = lax.axis_index("c")
        sid = lax.axis_index("s")
        sem_g = (sem_g0, sem_g1)
        sem_e = (sem_e0, sem_e1)

        def zfill(i, _):
            r = i // (HD // VL)
            c = lax.rem(i, HD // VL) * VL
            h_v[r, pl.ds(c, VL)] = jnp.zeros((VL,), jnp.float32)
            return 0

        lax.fori_loop(0, BLK * HD // VL, zfill, 0)

        def czfill(r, _):
            cz_v[r, pl.ds(0, CW)] = jnp.zeros((CW,), jnp.float32)
            return 0

        lax.fori_loop(0, BLK, czfill, 0)

        def onefill(r, _):
            ones_v[r, pl.ds(0, CW)] = jnp.ones((CW,), jnp.float32)
            return 0

        lax.fori_loop(0, BLK, onefill, 0)

        for k in range(RPT // BLK):
            pltpu.sync_copy(h_v, agg_sh.at[pl.ds(sid * RPT + k * BLK, BLK)])

        @pl.when(cid == 0)
        def _():
            for k in range(RPT // BLK):
                pltpu.sync_copy(cz_v, cnt_sh.at[pl.ds(sid * RPT + k * BLK, BLK)])

        pltpu.sync_copy(sidx_hbm.at[sid], sidx_v)
        pltpu.sync_copy(didx_hbm.at[sid], didx_v)
        plsc.subcore_barrier()

        ebase = cid * (E // 2) + sid * (EPT // 2)
        coff = cid * HD

        def issue(jb, b):
            pltpu.async_copy(src_proj_hbm.at[sidx_v.at[jb]], rows_v.at[b],
                             sem_g[b])
            pltpu.async_copy(edge_proj_hbm.at[pl.ds(ebase + jb * PBLK, PBLK)],
                             ep_v.at[b], sem_e[b])

        for b in range(2):
            issue(b, b)

        def group(t, _):
            for b in range(2):
                jb = 2 * t + b
                pltpu.make_async_copy(
                    src_proj_hbm.at[pl.ds(0, BLK)], rows_v.at[b],
                    sem_g[b]).wait()
                pltpu.make_async_copy(
                    edge_proj_hbm.at[pl.ds(0, PBLK)], ep_v.at[b],
                    sem_e[b]).wait()

                def comp(j, _):
                    for p in range(2):
                        r = 2 * j + p
                        for cc in range(HD // VL):
                            h_v[r, pl.ds(cc * VL, VL)] = jnp.maximum(
                                rows_v[b, r, pl.ds(coff + cc * VL, VL)]
                                + ep_v[b, j, pl.ds(p * HD + cc * VL, VL)],
                                0.0,
                            )
                    return 0

                lax.fori_loop(0, PBLK, comp, 0)

                @pl.when(jb + 2 < NBLK)
                def _():
                    issue(jb + 2, b)

                pltpu.sync_copy(h_v, agg_sh.at[didx_v.at[jb]], add=True)

                @pl.when(cid == 0)
                def _():
                    pltpu.sync_copy(ones_v, cnt_sh.at[didx_v.at[jb]], add=True)

            return 0

        lax.fori_loop(0, NBLK // 2, group, 0)
        plsc.subcore_barrier()

        for k in range(RPT // BLK):
            off = sid * RPT + k * BLK
            pltpu.sync_copy(agg_sh.at[pl.ds(off, BLK)],
                            agg_out.at[cid, pl.ds(off, BLK)])

        @pl.when(cid == 0)
        def _():
            pltpu.sync_copy(cnt_sh.at[pl.ds(sid * RPT, RPT)],
                            cnt_out.at[pl.ds(sid * RPT, RPT)])

    return body(src_proj, edge_projp, src_idx, dst_idx)


def _post_body(p_ref, c_ref, dstf_ref, w2_ref, b2_ref, u1_ref, c1_ref,
               u2_ref, c2_ref, o_ref):
    agg_h = jnp.concatenate([p_ref[0], p_ref[1]], axis=-1)
    cnt = c_ref[:, 0:1]
    aggregated = (
        jnp.dot(agg_h, w2_ref[...], preferred_element_type=jnp.float32)
        + cnt * b2_ref[...]
    )
    upd = jnp.concatenate([dstf_ref[...], aggregated], axis=-1)
    g = jnp.maximum(
        jnp.dot(upd, u1_ref[...], preferred_element_type=jnp.float32)
        + c1_ref[...],
        0.0,
    )
    o_ref[...] = (
        jnp.dot(g, u2_ref[...], preferred_element_type=jnp.float32) + c2_ref[...]
    )


def _post(agg_p, cnt_p, dst_features, W2, b2, U1, c1, U2, c2):
    bm = 1000
    return pl.pallas_call(
        _post_body,
        grid=(N_DST // bm,),
        in_specs=[
            pl.BlockSpec((NC, bm, HD), lambda i: (0, i, 0)),
            pl.BlockSpec((bm, CW), lambda i: (i, 0)),
            pl.BlockSpec((bm, D), lambda i: (i, 0)),
            pl.BlockSpec((D, D), lambda i: (0, 0)),
            pl.BlockSpec((1, D), lambda i: (0, 0)),
            pl.BlockSpec((2 * D, D), lambda i: (0, 0)),
            pl.BlockSpec((1, D), lambda i: (0, 0)),
            pl.BlockSpec((D, D), lambda i: (0, 0)),
            pl.BlockSpec((1, D), lambda i: (0, 0)),
        ],
        out_specs=pl.BlockSpec((bm, D), lambda i: (i, 0)),
        out_shape=jax.ShapeDtypeStruct((N_DST, D), jnp.float32),
    )(agg_p, cnt_p, dst_features, W2, b2.reshape(1, -1), U1,
      c1.reshape(1, -1), U2, c2.reshape(1, -1))


def kernel(src_features, dst_features, edge_index, edge_features,
           W1, b1, W2, b2, U1, c1, U2, c2):
    ei = edge_index.astype(jnp.int32)
    src_idx = ei[0].reshape(NS, NBLK, BLK)
    dst_idx = ei[1].reshape(NS, NBLK, BLK)
    src_proj = _project_bias(src_features, W1[:D], b1, 1000)
    edge_proj2 = _project_split(edge_features, W1[D:], 4000)
    edge_projp = edge_proj2.reshape(NC * E // 2, D)
    agg_p, cnt_p = _sc_aggregate(src_proj, edge_projp, src_idx, dst_idx)
    return _post(agg_p, cnt_p, dst_features, W2, b2, U1, c1, U2, c2)


# speedup vs baseline: 2.1668x; 2.1668x over previous
import functools

import jax
import jax.numpy as jnp
from jax import lax
from jax.experimental import pallas as pl
from jax.experimental.pallas import tpu as pltpu
from jax.experimental.pallas import tpu_sc as plsc

N_SRC = 10000
N_DST = 10000
N_PAD = 10240
E = 320000
D = 128
NC = 2
NS = 16
EPT = E // (NC * NS)
BLK = 40
NBLK = EPT // BLK
RPT = N_PAD // NS
CW = 8
VL = 16


def _mm_bias_body(x_ref, w_ref, b_ref, o_ref):
    o_ref[...] = (
        jnp.dot(x_ref[...], w_ref[...], preferred_element_type=jnp.float32)
        + b_ref[...]
    )


def _project_bias(x, w, b, bm):
    m, k = x.shape
    n = w.shape[1]
    return pl.pallas_call(
        _mm_bias_body,
        grid=(m // bm,),
        in_specs=[
            pl.BlockSpec((bm, k), lambda i: (i, 0)),
            pl.BlockSpec((k, n), lambda i: (0, 0)),
            pl.BlockSpec((1, n), lambda i: (0, 0)),
        ],
        out_specs=pl.BlockSpec((bm, n), lambda i: (i, 0)),
        out_shape=jax.ShapeDtypeStruct((m, n), jnp.float32),
    )(x, w, b.reshape(1, -1))


def _mm_body(x_ref, w_ref, o_ref):
    o_ref[...] = jnp.dot(x_ref[...], w_ref[...],
                         preferred_element_type=jnp.float32)


def _project(x, w, bm):
    m, k = x.shape
    n = w.shape[1]
    return pl.pallas_call(
        _mm_body,
        grid=(m // bm,),
        in_specs=[
            pl.BlockSpec((bm, k), lambda i: (i, 0)),
            pl.BlockSpec((k, n), lambda i: (0, 0)),
        ],
        out_specs=pl.BlockSpec((bm, n), lambda i: (i, 0)),
        out_shape=jax.ShapeDtypeStruct((m, n), jnp.float32),
    )(x, w)


def _sc_aggregate(src_proj, edge_proj, src_idx, dst_idx):
    mesh = plsc.VectorSubcoreMesh(core_axis_name="c", subcore_axis_name="s")

    @functools.partial(
        pl.kernel,
        mesh=mesh,
        compiler_params=pltpu.CompilerParams(use_tc_tiling_on_sc=False),
        out_type=[
            jax.ShapeDtypeStruct((NC, N_PAD, D), jnp.float32),
            jax.ShapeDtypeStruct((NC, N_PAD, CW), jnp.float32),
        ],
        scratch_types=[
            pltpu.VMEM((NBLK, BLK), jnp.int32),
            pltpu.VMEM((NBLK, BLK), jnp.int32),
            pltpu.VMEM((2, BLK, D), jnp.float32),
            pltpu.VMEM((2, BLK, D), jnp.float32),
            pltpu.VMEM((BLK, CW), jnp.float32),
            pltpu.VMEM((BLK, CW), jnp.float32),
            pltpu.SemaphoreType.DMA,
            pltpu.SemaphoreType.DMA,
            pltpu.SemaphoreType.DMA,
            pltpu.SemaphoreType.DMA,
            pltpu.VMEM_SHARED((N_PAD, D), jnp.float32),
            pltpu.VMEM_SHARED((N_PAD, CW), jnp.float32),
        ],
    )
    def body(src_proj_hbm, edge_proj_hbm, sidx_hbm, didx_hbm,
             agg_out, cnt_out,
             sidx_v, didx_v, rows_v, ep_v, cz_v, ones_v,
             sem_g0, sem_g1, sem_e0, sem_e1, agg_sh, cnt_sh):
        cid = lax.axis_index("c")
        sid = lax.axis_index("s")
        sem_g = (sem_g0, sem_g1)
        sem_e = (sem_e0, sem_e1)

        def zfill(i, _):
            r = i // (D // VL)
            c = lax.rem(i, D // VL) * VL
            rows_v[0, r, pl.ds(c, VL)] = jnp.zeros((VL,), jnp.float32)
            return 0

        lax.fori_loop(0, BLK * D // VL, zfill, 0)

        def czfill(r, _):
            cz_v[r, pl.ds(0, CW)] = jnp.zeros((CW,), jnp.float32)
            ones_v[r, pl.ds(0, CW)] = jnp.ones((CW,), jnp.float32)
            return 0

        lax.fori_loop(0, BLK, czfill, 0)

        for k in range(RPT // BLK):
            pltpu.sync_copy(rows_v.at[0],
                            agg_sh.at[pl.ds(sid * RPT + k * BLK, BLK)])
            pltpu.sync_copy(cz_v, cnt_sh.at[pl.ds(sid * RPT + k * BLK, BLK)])

        pltpu.sync_copy(sidx_hbm.at[cid, sid], sidx_v)
        pltpu.sync_copy(didx_hbm.at[cid, sid], didx_v)
        plsc.subcore_barrier()

        ebase = cid * (E // NC) + sid * EPT

        def issue(jb, b):
            pltpu.async_copy(src_proj_hbm.at[sidx_v.at[jb]], rows_v.at[b],
                             sem_g[b])
            pltpu.async_copy(edge_proj_hbm.at[pl.ds(ebase + jb * BLK, BLK)],
                             ep_v.at[b], sem_e[b])

        for b in range(2):
            issue(b, b)

        def group(t, _):
            for b in range(2):
                jb = 2 * t + b
                pltpu.make_async_copy(
                    src_proj_hbm.at[pl.ds(0, BLK)], rows_v.at[b],
                    sem_g[b]).wait()
                pltpu.make_async_copy(
                    edge_proj_hbm.at[pl.ds(0, BLK)], ep_v.at[b],
                    sem_e[b]).wait()

                def comp(j, _):
                    for p in range(2):
                        r = 2 * j + p
                        for cc in range(D // VL):
                            rows_v[b, r, pl.ds(cc * VL, VL)] = jnp.maximum(
                                rows_v[b, r, pl.ds(cc * VL, VL)]
                                + ep_v[b, j * 2 + p, pl.ds(cc * VL, VL)],
                                0.0,
                            )
                    return 0

                lax.fori_loop(0, BLK // 2, comp, 0)

                pltpu.sync_copy(rows_v.at[b], agg_sh.at[didx_v.at[jb]],
                                add=True)
                pltpu.sync_copy(ones_v, cnt_sh.at[didx_v.at[jb]], add=True)

                @pl.when(jb + 2 < NBLK)
                def _():
                    issue(jb + 2, b)

            return 0

        lax.fori_loop(0, NBLK // 2, group, 0)
        plsc.subcore_barrier()

        for k in range(RPT // BLK):
            off = sid * RPT + k * BLK
            pltpu.sync_copy(agg_sh.at[pl.ds(off, BLK)],
                            agg_out.at[cid, pl.ds(off, BLK)])

        pltpu.sync_copy(cnt_sh.at[pl.ds(sid * RPT, RPT)],
                        cnt_out.at[cid, pl.ds(sid * RPT, RPT)])

    return body(src_proj, edge_proj, src_idx, dst_idx)


def _post_body(p_ref, c_ref, dstf_ref, w2_ref, b2_ref, u1_ref, c1_ref,
               u2_ref, c2_ref, o_ref):
    agg_h = p_ref[0] + p_ref[1]
    cnt = c_ref[0, :, 0:1] + c_ref[1, :, 0:1]
    aggregated = (
        jnp.dot(agg_h, w2_ref[...], preferred_element_type=jnp.float32)
        + cnt * b2_ref[...]
    )
    upd = jnp.concatenate([dstf_ref[...], aggregated], axis=-1)
    g = jnp.maximum(
        jnp.dot(upd, u1_ref[...], preferred_element_type=jnp.float32)
        + c1_ref[...],
        0.0,
    )
    o_ref[...] = (
        jnp.dot(g, u2_ref[...], preferred_element_type=jnp.float32) + c2_ref[...]
    )


def _post(agg_p, cnt_p, dst_features, W2, b2, U1, c1, U2, c2):
    bm = 1000
    return pl.pallas_call(
        _post_body,
        grid=(N_DST // bm,),
        in_specs=[
            pl.BlockSpec((NC, bm, D), lambda i: (0, i, 0)),
            pl.BlockSpec((NC, bm, CW), lambda i: (0, i, 0)),
            pl.BlockSpec((bm, D), lambda i: (i, 0)),
            pl.BlockSpec((D, D), lambda i: (0, 0)),
            pl.BlockSpec((1, D), lambda i: (0, 0)),
            pl.BlockSpec((2 * D, D), lambda i: (0, 0)),
            pl.BlockSpec((1, D), lambda i: (0, 0)),
            pl.BlockSpec((D, D), lambda i: (0, 0)),
            pl.BlockSpec((1, D), lambda i: (0, 0)),
        ],
        out_specs=pl.BlockSpec((bm, D), lambda i: (i, 0)),
        out_shape=jax.ShapeDtypeStruct((N_DST, D), jnp.float32),
    )(agg_p, cnt_p, dst_features, W2, b2.reshape(1, -1), U1,
      c1.reshape(1, -1), U2, c2.reshape(1, -1))


def kernel(src_features, dst_features, edge_index, edge_features,
           W1, b1, W2, b2, U1, c1, U2, c2):
    ei = edge_index.astype(jnp.int32)
    src_idx = ei[0].reshape(NC, NS, NBLK, BLK)
    dst_idx = ei[1].reshape(NC, NS, NBLK, BLK)
    src_proj = _project_bias(src_features, W1[:D], b1, 1000)
    edge_proj = _project(edge_features, W1[D:], 4000)
    agg_p, cnt_p = _sc_aggregate(src_proj, edge_proj, src_idx, dst_idx)
    return _post(agg_p, cnt_p, dst_features, W2, b2, U1, c1, U2, c2)


# speedup vs baseline: 2.1682x; 1.0006x over previous
import functools

import jax
import jax.numpy as jnp
from jax import lax
from jax.experimental import pallas as pl
from jax.experimental.pallas import tpu as pltpu
from jax.experimental.pallas import tpu_sc as plsc

N_SRC = 10000
N_DST = 10000
N_PAD = 10240
E = 320000
D = 128
NC = 2
NS = 16
EPT = E // (NC * NS)
BLK = 40
NBLK = EPT // BLK
RPT = N_PAD // NS
CW = 8
VL = 16


def _mm_bias_body(x_ref, w_ref, b_ref, o_ref):
    o_ref[...] = (
        jnp.dot(x_ref[...], w_ref[...], preferred_element_type=jnp.float32)
        + b_ref[...]
    )


def _project_bias(x, w, b, bm):
    m, k = x.shape
    n = w.shape[1]
    return pl.pallas_call(
        _mm_bias_body,
        grid=(m // bm,),
        in_specs=[
            pl.BlockSpec((bm, k), lambda i: (i, 0)),
            pl.BlockSpec((k, n), lambda i: (0, 0)),
            pl.BlockSpec((1, n), lambda i: (0, 0)),
        ],
        out_specs=pl.BlockSpec((bm, n), lambda i: (i, 0)),
        out_shape=jax.ShapeDtypeStruct((m, n), jnp.float32),
    )(x, w, b.reshape(1, -1))


def _mm_body(x_ref, w_ref, o_ref):
    o_ref[...] = jnp.dot(x_ref[...], w_ref[...],
                         preferred_element_type=jnp.float32)


def _project(x, w, bm):
    m, k = x.shape
    n = w.shape[1]
    return pl.pallas_call(
        _mm_body,
        grid=(m // bm,),
        in_specs=[
            pl.BlockSpec((bm, k), lambda i: (i, 0)),
            pl.BlockSpec((k, n), lambda i: (0, 0)),
        ],
        out_specs=pl.BlockSpec((bm, n), lambda i: (i, 0)),
        out_shape=jax.ShapeDtypeStruct((m, n), jnp.float32),
    )(x, w)


def _sc_aggregate(src_proj, edge_proj, src_idx, dst_idx):
    mesh = plsc.VectorSubcoreMesh(core_axis_name="c", subcore_axis_name="s")

    @functools.partial(
        pl.kernel,
        mesh=mesh,
        compiler_params=pltpu.CompilerParams(use_tc_tiling_on_sc=False),
        out_type=[
            jax.ShapeDtypeStruct((NC, N_PAD, D), jnp.float32),
            jax.ShapeDtypeStruct((NC, N_PAD, CW), jnp.float32),
        ],
        scratch_types=[
            pltpu.VMEM((EPT,), jnp.int32),
            pltpu.VMEM((NBLK, BLK), jnp.int32),
            pltpu.VMEM((2, BLK, D), jnp.float32),
            pltpu.VMEM((2, BLK, D), jnp.float32),
            pltpu.VMEM((BLK, CW), jnp.float32),
            pltpu.VMEM((BLK, CW), jnp.float32),
            pltpu.SemaphoreType.DMA,
            pltpu.SemaphoreType.DMA,
            pltpu.SemaphoreType.DMA,
            pltpu.SemaphoreType.DMA,
            pltpu.VMEM_SHARED((N_PAD, D), jnp.float32),
            pltpu.VMEM_SHARED((N_PAD, CW), jnp.float32),
        ],
    )
    def body(src_proj_hbm, edge_proj_hbm, sidx_hbm, didx_hbm,
             agg_out, cnt_out,
             sidx_v, didx_v, rows_v, ep_v, cz_v, ones_v,
             sem_g0, sem_g1, sem_e0, sem_e1, agg_sh, cnt_sh):
        cid = lax.axis_index("c")
        sid = lax.axis_index("s")
        sem_g = (sem_g0, sem_g1)
        sem_e = (sem_e0, sem_e1)

        def zfill(i, _):
            r = i // (D // VL)
            c = lax.rem(i, D // VL) * VL
            rows_v[0, r, pl.ds(c, VL)] = jnp.zeros((VL,), jnp.float32)
            return 0

        lax.fori_loop(0, BLK * D // VL, zfill, 0)

        def czfill(r, _):
            cz_v[r, pl.ds(0, CW)] = jnp.zeros((CW,), jnp.float32)
            ones_v[r, pl.ds(0, CW)] = jnp.ones((CW,), jnp.float32)
            return 0

        lax.fori_loop(0, BLK, czfill, 0)

        for k in range(RPT // BLK):
            pltpu.sync_copy(rows_v.at[0],
                            agg_sh.at[pl.ds(sid * RPT + k * BLK, BLK)])
            pltpu.sync_copy(cz_v, cnt_sh.at[pl.ds(sid * RPT + k * BLK, BLK)])

        ebase = cid * (E // NC) + sid * EPT
        pltpu.sync_copy(sidx_hbm.at[pl.ds(ebase, EPT)], sidx_v)
        pltpu.sync_copy(didx_hbm.at[cid, sid], didx_v)
        plsc.subcore_barrier()

        def issue(jb, b):
            pltpu.async_copy(
                src_proj_hbm.at[sidx_v.at[pl.ds(jb * BLK, BLK)]],
                rows_v.at[b], sem_g[b])
            pltpu.async_copy(edge_proj_hbm.at[pl.ds(ebase + jb * BLK, BLK)],
                             ep_v.at[b], sem_e[b])

        for b in range(2):
            issue(b, b)

        def group(t, _):
            for b in range(2):
                jb = 2 * t + b
                pltpu.make_async_copy(
                    src_proj_hbm.at[pl.ds(0, BLK)], rows_v.at[b],
                    sem_g[b]).wait()
                pltpu.make_async_copy(
                    edge_proj_hbm.at[pl.ds(0, BLK)], ep_v.at[b],
                    sem_e[b]).wait()

                def comp(j, _):
                    for p in range(2):
                        r = 2 * j + p
                        for cc in range(D // VL):
                            rows_v[b, r, pl.ds(cc * VL, VL)] = jnp.maximum(
                                rows_v[b, r, pl.ds(cc * VL, VL)]
                                + ep_v[b, j * 2 + p, pl.ds(cc * VL, VL)],
                                0.0,
                            )
                    return 0

                lax.fori_loop(0, BLK // 2, comp, 0)

                pltpu.sync_copy(rows_v.at[b], agg_sh.at[didx_v.at[jb]],
                                add=True)
                pltpu.sync_copy(ones_v, cnt_sh.at[didx_v.at[jb]], add=True)

                @pl.when(jb + 2 < NBLK)
                def _():
                    issue(jb + 2, b)

            return 0

        lax.fori_loop(0, NBLK // 2, group, 0)
        plsc.subcore_barrier()

        for k in range(RPT // BLK):
            off = sid * RPT + k * BLK
            pltpu.sync_copy(agg_sh.at[pl.ds(off, BLK)],
                            agg_out.at[cid, pl.ds(off, BLK)])

        pltpu.sync_copy(cnt_sh.at[pl.ds(sid * RPT, RPT)],
                        cnt_out.at[cid, pl.ds(sid * RPT, RPT)])

    return body(src_proj, edge_proj, src_idx, dst_idx)


def _post_body(p_ref, c_ref, dstf_ref, w2_ref, b2_ref, u1_ref, c1_ref,
               u2_ref, c2_ref, o_ref):
    agg_h = p_ref[0] + p_ref[1]
    cnt = c_ref[0, :, 0:1] + c_ref[1, :, 0:1]
    aggregated = (
        jnp.dot(agg_h, w2_ref[...], preferred_element_type=jnp.float32)
        + cnt * b2_ref[...]
    )
    upd = jnp.concatenate([dstf_ref[...], aggregated], axis=-1)
    g = jnp.maximum(
        jnp.dot(upd, u1_ref[...], preferred_element_type=jnp.float32)
        + c1_ref[...],
        0.0,
    )
    o_ref[...] = (
        jnp.dot(g, u2_ref[...], preferred_element_type=jnp.float32) + c2_ref[...]
    )


def _post(agg_p, cnt_p, dst_features, W2, b2, U1, c1, U2, c2):
    bm = 1000
    return pl.pallas_call(
        _post_body,
        grid=(N_DST // bm,),
        in_specs=[
            pl.BlockSpec((NC, bm, D), lambda i: (0, i, 0)),
            pl.BlockSpec((NC, bm, CW), lambda i: (0, i, 0)),
            pl.BlockSpec((bm, D), lambda i: (i, 0)),
            pl.BlockSpec((D, D), lambda i: (0, 0)),
            pl.BlockSpec((1, D), lambda i: (0, 0)),
            pl.BlockSpec((2 * D, D), lambda i: (0, 0)),
            pl.BlockSpec((1, D), lambda i: (0, 0)),
            pl.BlockSpec((D, D), lambda i: (0, 0)),
            pl.BlockSpec((1, D), lambda i: (0, 0)),
        ],
        out_specs=pl.BlockSpec((bm, D), lambda i: (i, 0)),
        out_shape=jax.ShapeDtypeStruct((N_DST, D), jnp.float32),
    )(agg_p, cnt_p, dst_features, W2, b2.reshape(1, -1), U1,
      c1.reshape(1, -1), U2, c2.reshape(1, -1))


def kernel(src_features, dst_features, edge_index, edge_features,
           W1, b1, W2, b2, U1, c1, U2, c2):
    ei = edge_index.astype(jnp.int32)
    src_idx = ei[0]
    dst_idx = ei[1].reshape(NC, NS, NBLK, BLK)
    src_proj = _project_bias(src_features, W1[:D], b1, 1000)
    edge_proj = _project(edge_features, W1[D:], 4000)
    agg_p, cnt_p = _sc_aggregate(src_proj, edge_proj, src_idx, dst_idx)
    return _post(agg_p, cnt_p, dst_features, W2, b2, U1, c1, U2, c2)
